# Initial kernel scaffold; baseline (speedup 1.0000x reference)
import jax
import jax.numpy as jnp
from jax.experimental import pallas as pl


def kernel(x, edge_index, W1l, b1l, W1r, W2l, b2l, W2r):
    raise NotImplementedError("write your pallas kernel here")


# speedup vs baseline: 5.9403x; 5.9403x over previous
import functools

import jax
import jax.numpy as jnp
from jax import lax
from jax.experimental import pallas as pl
from jax.experimental.pallas import tpu as pltpu
from jax.experimental.pallas import tpu_sc as plsc

N = 10000
D = 256
H = 128
E = 160000
NC = 2
NS = 16
C = 128
CH = 80
EPT = C * CH
E_PAD = EPT * NS
NPAD = 112
NROW = N + NPAD
RPT = NROW // NS
BN = 1000


def _sc_agg_body(with_cnt, *refs):
    if with_cnt:
        (hflat, srcp, dstp, zrows, z16, o16, agg, cnt,
         acc, cacc, sidx, didx, rows, ones, sem) = refs
    else:
        (hflat, srcp, dstp, zrows, agg,
         acc, sidx, didx, rows, sem) = refs
    cid = lax.axis_index("c")
    sid = lax.axis_index("s")
    base = sid * RPT

    pltpu.sync_copy(srcp.at[cid, sid], sidx)
    pltpu.sync_copy(dstp.at[sid], didx)
    pltpu.sync_copy(zrows, rows)
    for k in range(4):
        pltpu.sync_copy(rows, acc.at[pl.ds(base + k * C, C)])
    pltpu.sync_copy(rows.at[pl.ds(0, RPT - 4 * C)],
                    acc.at[pl.ds(base + 4 * C, RPT - 4 * C)])
    if with_cnt:
        @pl.when(cid == 0)
        def _():
            pltpu.sync_copy(z16, ones)
            for k in range(4):
                pltpu.sync_copy(ones, cacc.at[pl.ds(base + k * C, C)])
            pltpu.sync_copy(ones.at[pl.ds(0, RPT - 4 * C)],
                            cacc.at[pl.ds(base + 4 * C, RPT - 4 * C)])
            pltpu.sync_copy(o16, ones)

    plsc.subcore_barrier()

    def chunk(j, carry):
        pltpu.async_copy(hflat.at[sidx.at[j]], rows, sem).wait()
        pltpu.sync_copy(rows, acc.at[didx.at[j]], add=True)
        if with_cnt:
            @pl.when(cid == 0)
            def _():
                pltpu.sync_copy(ones, cacc.at[didx.at[j]], add=True)
        return carry

    lax.fori_loop(0, CH, chunk, 0)

    plsc.subcore_barrier()
    pltpu.sync_copy(acc.at[pl.ds(base, RPT)], agg.at[cid, pl.ds(base, RPT)])
    if with_cnt:
        @pl.when(cid == 0)
        def _():
            pltpu.sync_copy(cacc.at[pl.ds(base, RPT)], cnt.at[pl.ds(base, RPT)])


def _make_sc_agg(with_cnt):
    mesh = plsc.VectorSubcoreMesh(core_axis_name="c", subcore_axis_name="s",
                                  num_cores=NC, num_subcores=NS)
    out_type = (jax.ShapeDtypeStruct((NC, NROW, H), jnp.float32),)
    scratch = [
        pltpu.VMEM_SHARED((NROW, H), jnp.float32),
    ]
    if with_cnt:
        out_type = out_type + (jax.ShapeDtypeStruct((NROW, 16), jnp.float32),)
        scratch.append(pltpu.VMEM_SHARED((NROW, 16), jnp.float32))
    scratch += [
        pltpu.VMEM((CH, C), jnp.int32),
        pltpu.VMEM((CH, C), jnp.int32),
        pltpu.VMEM((C, H), jnp.float32),
    ]
    if with_cnt:
        scratch.append(pltpu.VMEM((C, 16), jnp.float32))
    scratch.append(pltpu.SemaphoreType.DMA)
    return pl.kernel(functools.partial(_sc_agg_body, with_cnt),
                     out_type=out_type, mesh=mesh, scratch_types=scratch,
                     compiler_params=pltpu.CompilerParams(
                         use_tc_tiling_on_sc=False))


_sc_agg_l1 = _make_sc_agg(True)
_sc_agg_l2 = _make_sc_agg(False)


def _tc_layer_body(relu, a_ref, c_ref, h_ref, wla_ref, wlb_ref, wr_ref,
                   b_ref, o_ref):
    r = 1.0 / jnp.maximum(c_ref[:, 0:1], 1.0)
    acc = jnp.dot(a_ref[0] * r, wla_ref[...],
                  preferred_element_type=jnp.float32)
    acc += jnp.dot(a_ref[1] * r, wlb_ref[...],
                   preferred_element_type=jnp.float32)
    acc += jnp.dot(h_ref[...], wr_ref[...],
                   preferred_element_type=jnp.float32)
    acc += b_ref[...]
    o_ref[...] = jnp.maximum(acc, 0.0) if relu else acc


def _tc_layer(agg, cnt, h, Wl, bl, Wr, relu):
    wla = Wl[:, :H].T
    wlb = Wl[:, H:].T
    wr = Wr.T
    grid = (N // BN,)
    return pl.pallas_call(
        functools.partial(_tc_layer_body, relu),
        grid=grid,
        in_specs=[
            pl.BlockSpec((NC, BN, H), lambda i: (0, i, 0)),
            pl.BlockSpec((BN, 16), lambda i: (i, 0)),
            pl.BlockSpec((BN, D), lambda i: (i, 0)),
            pl.BlockSpec((H, D), lambda i: (0, 0)),
            pl.BlockSpec((H, D), lambda i: (0, 0)),
            pl.BlockSpec((D, D), lambda i: (0, 0)),
            pl.BlockSpec((1, D), lambda i: (0, 0)),
        ],
        out_specs=pl.BlockSpec((BN, D), lambda i: (i, 0)),
        out_shape=jax.ShapeDtypeStruct((N, D), jnp.float32),
    )(agg, cnt, h, wla, wlb, wr, bl.reshape(1, D))


def kernel(x, edge_index, W1l, b1l, W1r, W2l, b2l, W2r):
    src = edge_index[0].astype(jnp.int32)
    dst = edge_index[1].astype(jnp.int32)
    npad_e = E_PAD - E
    pad = jnp.arange(npad_e, dtype=jnp.int32)
    src_p = jnp.concatenate([src, pad % N])
    dst_p = jnp.concatenate([dst, N + pad % NPAD])
    srcp = ((2 * src_p)[None, :] +
            jnp.array([[0], [1]], jnp.int32)).reshape(NC, NS, CH, C)
    dstp = dst_p.reshape(NS, CH, C)
    zrows = jnp.zeros((C, H), jnp.float32)
    z16 = jnp.zeros((C, 16), jnp.float32)
    o16 = jnp.ones((C, 16), jnp.float32)

    agg1, cnt = _sc_agg_l1(x.reshape(2 * N, H), srcp, dstp, zrows, z16, o16)
    h1 = _tc_layer(agg1, cnt, x, W1l, b1l, W1r, relu=True)
    (agg2,) = _sc_agg_l2(h1.reshape(2 * N, H), srcp, dstp, zrows)
    out = _tc_layer(agg2, cnt, h1, W2l, b2l, W2r, relu=False)
    return out
